# baseline (speedup 1.0000x reference)
import jax
import jax.numpy as jnp
from jax import lax
from jax.experimental import pallas as pl
from jax.experimental.pallas import tpu as pltpu, tpu_sc as plsc

VOCAB = 1000000
D = 64
B = 4096
S = 50
SP = S // 2

_info = plsc.get_sparse_core_info()
NC, NS = _info.num_cores, _info.num_subcores
NW = NC * NS
BLK = B // NW
CH = 2 * BLK
L = 16
NG = BLK // L
DU = 8
RING = 3


def _body(xg_hbm, tab_hbm, out_hbm, xv, pr, st, gsem, osem):
    wid = lax.axis_index("s") * NC + lax.axis_index("c")
    pltpu.sync_copy(xg_hbm.at[wid], xv)
    iota = lax.iota(jnp.int32, L)

    def fire_g(s2, ph):
        pltpu.async_copy(tab_hbm.at[xv.at[pl.ds(s2 * CH, CH)]], pr.at[ph], gsem.at[ph])

    def prime(s2, c):
        fire_g(s2, s2)
        return c

    lax.fori_loop(0, RING, prime, 0)

    def step(s2, c):
        ph = lax.rem(s2, RING)
        prbuf = pr.at[ph]
        pltpu.make_async_copy(tab_hbm.at[xv.at[pl.ds(0, CH)]], prbuf, gsem.at[ph]).wait()

        for k in range(2):
            stbuf = st.at[k]

            @pl.when(s2 >= 1)
            def _():
                pltpu.make_async_copy(
                    stbuf, out_hbm.at[0, :, wid], osem.at[k]
                ).wait()

            for g in range(NG):
                rows = k * BLK + g * L + iota

                @plsc.parallel_loop(0, D, step=1, unroll=DU)
                def dstep(d):
                    vals = plsc.load_gather(prbuf, [rows, iota * 0 + d])
                    stbuf[d // 8, lax.rem(d, 8), pl.ds(g * L, L)] = vals

            pltpu.async_copy(
                stbuf, out_hbm.at[2 * s2 + k, :, wid], osem.at[k]
            )

        @pl.when(s2 < SP - RING)
        def _():
            fire_g(s2 + RING, ph)

        return c

    lax.fori_loop(0, SP, step, 0)
    pltpu.make_async_copy(st.at[0], out_hbm.at[0, :, wid], osem.at[0]).wait()
    pltpu.make_async_copy(st.at[1], out_hbm.at[0, :, wid], osem.at[1]).wait()


def kernel(x, table):
    xg = (
        x.T.reshape(SP, 2, NW, BLK).transpose(2, 0, 1, 3).reshape(NW, SP * CH)
    )
    tab = jnp.pad(table, ((0, 0), (0, 2 * D - table.shape[1])))
    mesh = plsc.VectorSubcoreMesh(core_axis_name="c", subcore_axis_name="s")
    out5 = pl.kernel(
        _body,
        out_type=jax.ShapeDtypeStruct((S, D // 8, NW, 8, BLK), jnp.float32),
        mesh=mesh,
        scratch_types=[
            pltpu.VMEM((SP * CH,), jnp.int32),
            pltpu.VMEM((RING, CH, 2 * D), jnp.float32),
            pltpu.VMEM((2, D // 8, 8, BLK), jnp.float32),
            pltpu.SemaphoreType.DMA((RING,)),
            pltpu.SemaphoreType.DMA((2,)),
        ],
        compiler_params=pltpu.CompilerParams(needs_layout_passes=False),
    )(xg, tab)
    return out5.transpose(2, 4, 0, 1, 3).reshape(B, S, D)


# speedup vs baseline: 1.0043x; 1.0043x over previous
import jax
import jax.numpy as jnp
from jax import lax
from jax.experimental import pallas as pl
from jax.experimental.pallas import tpu as pltpu, tpu_sc as plsc

VOCAB = 1000000
D = 64
B = 4096
S = 50
SP = S // 2

_info = plsc.get_sparse_core_info()
NC, NS = _info.num_cores, _info.num_subcores
NW = NC * NS
BLK = B // NW
CH = 2 * BLK
L = 16
NG = BLK // L
DU = 16
RING = 3


def _body(xg_hbm, tab_hbm, out_hbm, xv, pr, st, gsem, osem):
    wid = lax.axis_index("s") * NC + lax.axis_index("c")
    pltpu.sync_copy(xg_hbm.at[wid], xv)
    iota = lax.iota(jnp.int32, L)
    ones = iota < L

    def fire_g(s2, ph):
        pltpu.async_copy(tab_hbm.at[xv.at[pl.ds(s2 * CH, CH)]], pr.at[ph], gsem.at[ph])

    def prime(s2, c):
        fire_g(s2, s2)
        return c

    lax.fori_loop(0, RING, prime, 0)

    def step(s2, c):
        ph = lax.rem(s2, RING)
        prbuf = pr.at[ph]
        pltpu.make_async_copy(tab_hbm.at[xv.at[pl.ds(0, CH)]], prbuf, gsem.at[ph]).wait()

        for k in range(2):
            stbuf = st.at[k]

            @pl.when(s2 >= 1)
            def _():
                pltpu.make_async_copy(
                    stbuf, out_hbm.at[0, :, wid], osem.at[k]
                ).wait()

            for g in range(NG):
                rows = k * BLK + g * L + iota

                @plsc.parallel_loop(0, D, step=1, unroll=DU)
                def dstep(d):
                    vals = plsc.load_gather(prbuf, [rows, iota * 0 + d], mask=ones)
                    stbuf[d // 8, lax.rem(d, 8), pl.ds(g * L, L)] = vals

            pltpu.async_copy(
                stbuf, out_hbm.at[2 * s2 + k, :, wid], osem.at[k]
            )

        @pl.when(s2 < SP - RING)
        def _():
            fire_g(s2 + RING, ph)

        return c

    lax.fori_loop(0, SP, step, 0)
    pltpu.make_async_copy(st.at[0], out_hbm.at[0, :, wid], osem.at[0]).wait()
    pltpu.make_async_copy(st.at[1], out_hbm.at[0, :, wid], osem.at[1]).wait()


def kernel(x, table):
    xg = (
        x.T.reshape(SP, 2, NW, BLK).transpose(2, 0, 1, 3).reshape(NW, SP * CH)
    )
    tab = jnp.pad(table, ((0, 0), (0, 2 * D - table.shape[1])))
    mesh = plsc.VectorSubcoreMesh(core_axis_name="c", subcore_axis_name="s")
    out5 = pl.kernel(
        _body,
        out_type=jax.ShapeDtypeStruct((S, D // 8, NW, 8, BLK), jnp.float32),
        mesh=mesh,
        scratch_types=[
            pltpu.VMEM((SP * CH,), jnp.int32),
            pltpu.VMEM((RING, CH, 2 * D), jnp.float32),
            pltpu.VMEM((2, D // 8, 8, BLK), jnp.float32),
            pltpu.SemaphoreType.DMA((RING,)),
            pltpu.SemaphoreType.DMA((2,)),
        ],
        compiler_params=pltpu.CompilerParams(needs_layout_passes=False),
    )(xg, tab)
    return out5.transpose(2, 4, 0, 1, 3).reshape(B, S, D)
